# baseline (speedup 1.0000x reference)
import functools

import jax
import jax.numpy as jnp
from jax import lax
from jax.experimental import pallas as pl
from jax.experimental.pallas import tpu as pltpu
from jax.experimental.pallas import tpu_sc as plsc

B = 32
L = 2048
IN_CH = 8
PATCH = 16
D = 64
VOCAB = 8192
N = L // PATCH
ROWS = B * N
PF = PATCH * IN_CH

BLK = 512
GRID = ROWS // BLK
CHUNK = 512
NCHUNK = VOCAB // CHUNK

_NC = 2
_NS = 16
_NW = _NC * _NS

HALF = ROWS // 2
GRIDH = HALF // BLK
_BPW = HALF // _NW


SUB = 8
NSLICE = VOCAB // SUB


def _tc_body(xt_ref, w_ref, b_ref, cb_ref, embt_ref, ids_ref, loss_ref,
             acc_ref):
    i = pl.program_id(0)
    embt = lax.dot_general(w_ref[...], xt_ref[...],
                           (((1,), (0,)), ((), ()))) + b_ref[...]
    embt_ref[...] = embt

    cb = cb_ref[...]
    e2 = jnp.sum(embt * embt, axis=0, keepdims=True)
    c2 = jnp.sum(cb * cb, axis=1, keepdims=True)
    a2 = lax.dot_general(cb, embt + embt,
                         (((1,), (0,)), ((), ())))

    m = None
    kf = None
    for k in range(NSLICE):
        sl = slice(k * SUB, (k + 1) * SUB)
        d = e2 - a2[sl, :] + c2[sl, :]
        if k == 0:
            m = d
            kf = jnp.zeros_like(d)
        else:
            upd = d < m
            m = jnp.minimum(m, d)
            kf = jnp.where(upd, jnp.float32(k * SUB), kf)

    subl = lax.broadcasted_iota(jnp.int32, (SUB, 1), 0).astype(jnp.float32)
    idc = kf + subl
    while m.shape[0] > 1:
        h = m.shape[0] // 2
        va, vb = m[:h], m[h:]
        ia, ib = idc[:h], idc[h:]
        lt = va < vb
        eq = va == vb
        m = jnp.minimum(va, vb)
        idc = jnp.where(lt, ia, jnp.where(eq, jnp.minimum(ia, ib), ib))
    minval = m
    ids_ref[0, 0, :] = idc[0].astype(jnp.int32)

    @pl.when(i == 0)
    def _():
        acc_ref[...] = minval

    @pl.when(i > 0)
    def _():
        acc_ref[...] += minval

    @pl.when(i == GRIDH - 1)
    def _():
        loss_ref[...] = jnp.sum(acc_ref[...])[None, None]


_DPAD = 128


@functools.cache
def _make_sc_gather():
    mesh = plsc.VectorSubcoreMesh(core_axis_name="c", subcore_axis_name="s")

    @functools.partial(
        pl.kernel,
        mesh=mesh,
        out_type=jax.ShapeDtypeStruct((HALF, _DPAD), jnp.float32),
        scratch_types=[
            pltpu.VMEM((_BPW,), jnp.int32),
            pltpu.VMEM((_BPW, _DPAD), jnp.float32),
            pltpu.SemaphoreType.DMA,
        ],
    )
    def _sc_gather(cb_hbm, idx_hbm, out_hbm, idx_v, rows_v, sem):
        wid = lax.axis_index("s") * _NC + lax.axis_index("c")
        base = wid * _BPW
        pltpu.sync_copy(idx_hbm.at[pl.ds(base, _BPW)], idx_v)
        pltpu.async_copy(cb_hbm.at[idx_v], rows_v, sem).wait()
        pltpu.sync_copy(rows_v, out_hbm.at[pl.ds(base, _BPW)])

    return _sc_gather


def _tc_half(xt_h, W, b2, codebook):
    return pl.pallas_call(
        _tc_body,
        grid=(GRIDH,),
        in_specs=[
            pl.BlockSpec((PF, BLK), lambda i: (0, i)),
            pl.BlockSpec((D, PF), lambda i: (0, 0)),
            pl.BlockSpec((D, 1), lambda i: (0, 0)),
            pl.BlockSpec((VOCAB, D), lambda i: (0, 0)),
        ],
        out_specs=[
            pl.BlockSpec((D, BLK), lambda i: (0, i)),
            pl.BlockSpec((1, 1, BLK), lambda i: (i, 0, 0)),
            pl.BlockSpec((1, 1), lambda i: (0, 0)),
        ],
        out_shape=[
            jax.ShapeDtypeStruct((D, HALF), jnp.float32),
            jax.ShapeDtypeStruct((GRIDH, 1, BLK), jnp.int32),
            jax.ShapeDtypeStruct((1, 1), jnp.float32),
        ],
        scratch_shapes=[pltpu.VMEM((1, BLK), jnp.float32)],
    )(xt_h, W, b2, codebook)


def kernel(x, W, b, codebook):
    xt = x.reshape(ROWS, PF).T
    b2 = b.reshape(D, 1)
    cb_pad = jnp.concatenate(
        [codebook, jnp.zeros((VOCAB, _DPAD - D), jnp.float32)], axis=1)
    gather = _make_sc_gather()

    embts, idss, quants, losses = [], [], [], []
    for h in range(2):
        embt_h, ids_h, loss_h = _tc_half(
            xt[:, h * HALF:(h + 1) * HALF], W, b2, codebook)
        embts.append(embt_h)
        idss.append(ids_h.reshape(HALF))
        quants.append(gather(cb_pad, idss[h])[:, :D])
        losses.append(loss_h[0, 0])

    ids_flat = jnp.concatenate(idss)
    emb = jnp.concatenate(embts, axis=1).T
    quantized = jnp.concatenate(quants)

    token_ids = ids_flat.reshape(B, N)
    patch_emb = emb.reshape(B, N, D)
    quantized_st = quantized.reshape(B, N, D)
    vq_loss = (2.0 / (ROWS * D)) * (losses[0] + losses[1])
    return (token_ids, patch_emb, quantized_st, vq_loss)


# speedup vs baseline: 1.0462x; 1.0462x over previous
import functools

import jax
import jax.numpy as jnp
from jax import lax
from jax.experimental import pallas as pl
from jax.experimental.pallas import tpu as pltpu
from jax.experimental.pallas import tpu_sc as plsc

B = 32
L = 2048
IN_CH = 8
PATCH = 16
D = 64
VOCAB = 8192
N = L // PATCH
ROWS = B * N
PF = PATCH * IN_CH

BLK = 1024
GRID = ROWS // BLK
CHUNK = 512
NCHUNK = VOCAB // CHUNK

_NC = 2
_NS = 16
_NW = _NC * _NS

_BPW = ROWS // _NW


SUB = 8
NSLICE = VOCAB // SUB


def _tc_body(xt_ref, w_ref, b_ref, cb_ref, embt_ref, ids_ref, loss_ref,
             acc_ref):
    i = pl.program_id(0)
    embt = lax.dot_general(w_ref[...], xt_ref[...],
                           (((1,), (0,)), ((), ()))) + b_ref[...]
    embt_ref[...] = embt

    cb = cb_ref[...]
    e2 = jnp.sum(embt * embt, axis=0, keepdims=True)
    c2 = jnp.sum(cb * cb, axis=1, keepdims=True)
    a2 = lax.dot_general(cb, embt + embt,
                         (((1,), (0,)), ((), ())))

    m = None
    kf = None
    for k in range(NSLICE):
        sl = slice(k * SUB, (k + 1) * SUB)
        d = e2 - a2[sl, :] + c2[sl, :]
        if k == 0:
            m = d
            kf = jnp.zeros_like(d)
        else:
            upd = d < m
            m = jnp.minimum(m, d)
            kf = jnp.where(upd, jnp.float32(k * SUB), kf)

    subl = lax.broadcasted_iota(jnp.int32, (SUB, 1), 0).astype(jnp.float32)
    idc = kf + subl
    while m.shape[0] > 1:
        h = m.shape[0] // 2
        va, vb = m[:h], m[h:]
        ia, ib = idc[:h], idc[h:]
        lt = va < vb
        eq = va == vb
        m = jnp.minimum(va, vb)
        idc = jnp.where(lt, ia, jnp.where(eq, jnp.minimum(ia, ib), ib))
    minval = m
    ids_ref[0, 0, :] = idc[0].astype(jnp.int32)

    @pl.when(i == 0)
    def _():
        acc_ref[...] = minval

    @pl.when(i > 0)
    def _():
        acc_ref[...] += minval

    @pl.when(i == GRID - 1)
    def _():
        loss_ref[...] = jnp.sum(acc_ref[...])[None, None]


_DPAD = 128


@functools.cache
def _make_sc_gather():
    mesh = plsc.VectorSubcoreMesh(core_axis_name="c", subcore_axis_name="s")

    @functools.partial(
        pl.kernel,
        mesh=mesh,
        out_type=jax.ShapeDtypeStruct((ROWS, _DPAD), jnp.float32),
        scratch_types=[
            pltpu.VMEM((_BPW,), jnp.int32),
            pltpu.VMEM((_BPW, _DPAD), jnp.float32),
            pltpu.SemaphoreType.DMA,
        ],
    )
    def _sc_gather(cb_hbm, idx_hbm, out_hbm, idx_v, rows_v, sem):
        wid = lax.axis_index("s") * _NC + lax.axis_index("c")
        base = wid * _BPW
        pltpu.sync_copy(idx_hbm.at[pl.ds(base, _BPW)], idx_v)
        pltpu.async_copy(cb_hbm.at[idx_v], rows_v, sem).wait()
        pltpu.sync_copy(rows_v, out_hbm.at[pl.ds(base, _BPW)])

    return _sc_gather


def kernel(x, W, b, codebook):
    xt = x.reshape(ROWS, PF).T
    b2 = b.reshape(D, 1)

    embt, ids, loss_raw = pl.pallas_call(
        _tc_body,
        grid=(GRID,),
        in_specs=[
            pl.BlockSpec((PF, BLK), lambda i: (0, i)),
            pl.BlockSpec((D, PF), lambda i: (0, 0)),
            pl.BlockSpec((D, 1), lambda i: (0, 0)),
            pl.BlockSpec((VOCAB, D), lambda i: (0, 0)),
        ],
        out_specs=[
            pl.BlockSpec((D, BLK), lambda i: (0, i)),
            pl.BlockSpec((1, 1, BLK), lambda i: (i, 0, 0)),
            pl.BlockSpec((1, 1), lambda i: (0, 0)),
        ],
        out_shape=[
            jax.ShapeDtypeStruct((D, ROWS), jnp.float32),
            jax.ShapeDtypeStruct((GRID, 1, BLK), jnp.int32),
            jax.ShapeDtypeStruct((1, 1), jnp.float32),
        ],
        scratch_shapes=[pltpu.VMEM((1, BLK), jnp.float32)],
    )(xt, W, b2, codebook)

    emb = embt.T
    ids_flat = ids.reshape(ROWS)
    cb_pad = jnp.concatenate(
        [codebook, jnp.zeros((VOCAB, _DPAD - D), jnp.float32)], axis=1)
    quantized = _make_sc_gather()(cb_pad, ids_flat)[:, :D]

    token_ids = ids_flat.reshape(B, N)
    patch_emb = emb.reshape(B, N, D)
    quantized_st = quantized.reshape(B, N, D)
    vq_loss = (2.0 / (ROWS * D)) * loss_raw[0, 0]
    return (token_ids, patch_emb, quantized_st, vq_loss)
